# baseline (speedup 1.0000x reference)
import functools
from math import sqrt

import jax
import jax.numpy as jnp
from jax import lax
from jax.experimental import pallas as pl
from jax.experimental.pallas import tpu as pltpu
from jax.experimental.pallas import tpu_sc as plsc

_VOCAB = 100000
_D = 1024
_SCALE = sqrt(_D)

_L = 16
_C = 16
_NBUF = 3
_SLICES = _D // _L


def _make_sc_gather(B):
    info = plsc.get_sparse_core_info()
    NC, NS = info.num_cores, info.num_subcores
    NW = NC * NS
    per_w = B // NW
    NCH = per_w // _C
    mesh = plsc.VectorSubcoreMesh(core_axis_name="c", subcore_axis_name="s")

    scratch = [pltpu.VMEM((per_w,), jnp.int32)]
    scratch += [pltpu.VMEM((_C, _D), jnp.float32)
                for _ in range(_NBUF)]
    scratch += [pltpu.VMEM((_C, _D), jnp.float32)
                for _ in range(_NBUF)]
    scratch += [pltpu.SemaphoreType.DMA for _ in range(2 * _NBUF)]

    @functools.partial(
        pl.kernel,
        mesh=mesh,
        out_type=jax.ShapeDtypeStruct((B, _D), jnp.float32),
        scratch_types=scratch,
    )
    def embed(lut_hbm, idx_hbm, out_hbm, idx_v, *bufs):
        wid = lax.axis_index("s") * NC + lax.axis_index("c")
        base = wid * per_w
        ins = bufs[:_NBUF]
        obs = bufs[_NBUF:2 * _NBUF]
        gss = bufs[2 * _NBUF:3 * _NBUF]
        oss = bufs[3 * _NBUF:]

        pltpu.sync_copy(idx_hbm.at[pl.ds(base, per_w)], idx_v)

        def gather_start(g, b):
            pltpu.async_copy(
                lut_hbm.at[idx_v.at[pl.ds(g * _C, _C)]], ins[b], gss[b])

        def gather_wait(g, b):
            pltpu.make_async_copy(
                lut_hbm.at[idx_v.at[pl.ds(g * _C, _C)]], ins[b], gss[b]).wait()

        def scatter_start(g, b):
            pltpu.async_copy(obs[b], out_hbm.at[pl.ds(base + g * _C, _C)], oss[b])

        def scatter_wait(g, b):
            pltpu.make_async_copy(
                obs[b], out_hbm.at[pl.ds(base + g * _C, _C)], oss[b]).wait()

        def scale(b):
            src, dst = ins[b], obs[b]

            def row(r, carry):
                for j in range(_SLICES):
                    dst[r, pl.ds(j * _L, _L)] = src[r, pl.ds(j * _L, _L)] * _SCALE
                return carry

            lax.fori_loop(0, _C, row, 0)

        for b in range(min(_NBUF, NCH)):
            gather_start(b, b)

        for g in range(NCH):
            b = g % _NBUF
            gather_wait(g, b)
            if g >= _NBUF:
                scatter_wait(g - _NBUF, b)
            scale(b)
            scatter_start(g, b)
            if g + _NBUF < NCH:
                gather_start(g + _NBUF, b)
        for g in range(max(0, NCH - _NBUF), NCH):
            scatter_wait(g, g % _NBUF)

    return embed


def kernel(input, lut):
    assert input.ndim == 2
    rows, cols = input.shape
    B = rows * cols
    idx = input.reshape(B)
    if idx.dtype != jnp.int32:
        idx = idx.astype(jnp.int32)
    out = _make_sc_gather(B)(lut, idx)
    return out.reshape(rows, cols, _D)


# speedup vs baseline: 1.1224x; 1.1224x over previous
import functools
from math import sqrt

import jax
import jax.numpy as jnp
from jax import lax
from jax.experimental import pallas as pl
from jax.experimental.pallas import tpu as pltpu
from jax.experimental.pallas import tpu_sc as plsc

_VOCAB = 100000
_D = 1024
_SCALE = sqrt(_D)

_L = 16
_C = 8
_NBUF = 6
_SLICES = _D // _L


def _make_sc_gather(B):
    info = plsc.get_sparse_core_info()
    NC, NS = info.num_cores, info.num_subcores
    NW = NC * NS
    per_w = B // NW
    NCH = per_w // _C
    mesh = plsc.VectorSubcoreMesh(core_axis_name="c", subcore_axis_name="s")

    scratch = [pltpu.VMEM((per_w,), jnp.int32)]
    scratch += [pltpu.VMEM((_C, _D), jnp.float32)
                for _ in range(_NBUF)]
    scratch += [pltpu.VMEM((_C, _D), jnp.float32)
                for _ in range(_NBUF)]
    scratch += [pltpu.SemaphoreType.DMA for _ in range(2 * _NBUF)]

    @functools.partial(
        pl.kernel,
        mesh=mesh,
        out_type=jax.ShapeDtypeStruct((B, _D), jnp.float32),
        scratch_types=scratch,
    )
    def embed(lut_hbm, idx_hbm, out_hbm, idx_v, *bufs):
        wid = lax.axis_index("s") * NC + lax.axis_index("c")
        base = wid * per_w
        ins = bufs[:_NBUF]
        obs = bufs[_NBUF:2 * _NBUF]
        gss = bufs[2 * _NBUF:3 * _NBUF]
        oss = bufs[3 * _NBUF:]

        pltpu.sync_copy(idx_hbm.at[pl.ds(base, per_w)], idx_v)

        def gather_start(g, b):
            pltpu.async_copy(
                lut_hbm.at[idx_v.at[pl.ds(g * _C, _C)]], ins[b], gss[b])

        def gather_wait(g, b):
            pltpu.make_async_copy(
                lut_hbm.at[idx_v.at[pl.ds(g * _C, _C)]], ins[b], gss[b]).wait()

        def scatter_start(g, b):
            pltpu.async_copy(obs[b], out_hbm.at[pl.ds(base + g * _C, _C)], oss[b])

        def scatter_wait(g, b):
            pltpu.make_async_copy(
                obs[b], out_hbm.at[pl.ds(base + g * _C, _C)], oss[b]).wait()

        def scale(b):
            src, dst = ins[b], obs[b]

            def row(r, carry):
                for j in range(_SLICES):
                    dst[r, pl.ds(j * _L, _L)] = src[r, pl.ds(j * _L, _L)] * _SCALE
                return carry

            lax.fori_loop(0, _C, row, 0)

        for b in range(min(_NBUF, NCH)):
            gather_start(b, b)

        def step(i, carry):
            g0 = i * _NBUF
            for b in range(_NBUF):
                g = g0 + b

                @pl.when(g < NCH)
                def _():
                    gather_wait(g, b)

                    @pl.when(g >= _NBUF)
                    def _():
                        scatter_wait(g - _NBUF, b)

                    scale(b)
                    scatter_start(g, b)

                    @pl.when(g + _NBUF < NCH)
                    def _():
                        gather_start(g + _NBUF, b)
            return carry

        lax.fori_loop(0, -(-NCH // _NBUF), step, 0)
        for g in range(max(0, NCH - _NBUF), NCH):
            scatter_wait(g, g % _NBUF)

    return embed


def kernel(input, lut):
    assert input.ndim == 2
    rows, cols = input.shape
    B = rows * cols
    idx = input.reshape(B)
    if idx.dtype != jnp.int32:
        idx = idx.astype(jnp.int32)
    out = _make_sc_gather(B)(lut, idx)
    return out.reshape(rows, cols, _D)
